# Initial kernel scaffold; baseline (speedup 1.0000x reference)
import jax
import jax.numpy as jnp
from jax.experimental import pallas as pl


def kernel(x, Wq, Wk, Wv, Wo):
    raise NotImplementedError("write your pallas kernel here")


# speedup vs baseline: 268.1320x; 268.1320x over previous
import jax
import jax.numpy as jnp
from jax.experimental import pallas as pl
from jax.experimental.pallas import tpu as pltpu

_CHUNK = 256

_F32 = jnp.float32


def _fastkv_kernel(x_ref, wq_ref, wk_ref, wv_ref, wo_ref, o_ref, s_ref):
    c = pl.program_id(1)

    @pl.when(c == 0)
    def _():
        s_ref[...] = jnp.zeros_like(s_ref)

    xb = x_ref[0]
    q = jax.lax.dot_general(xb, wq_ref[...], (((1,), (1,)), ((), ())),
                            preferred_element_type=_F32)
    k = jax.lax.dot_general(xb, wk_ref[...], (((1,), (1,)), ((), ())),
                            preferred_element_type=_F32)
    v = jax.lax.dot_general(xb, wv_ref[...], (((1,), (1,)), ((), ())),
                            preferred_element_type=_F32)

    y = jnp.dot(q, s_ref[...], preferred_element_type=_F32)

    a = jax.lax.dot_general(q, k, (((1,), (1,)), ((), ())),
                            preferred_element_type=_F32)
    i = jax.lax.broadcasted_iota(jnp.int32, a.shape, 0)
    j = jax.lax.broadcasted_iota(jnp.int32, a.shape, 1)
    a = jnp.where(i > j, a, 0.0)
    y = y + jnp.dot(a, v, preferred_element_type=_F32)

    s_ref[...] = s_ref[...] + jax.lax.dot_general(
        k, v, (((0,), (0,)), ((), ())), preferred_element_type=_F32)

    o_ref[0] = jax.lax.dot_general(y, wo_ref[...], (((1,), (1,)), ((), ())),
                                   preferred_element_type=_F32)


def kernel(x, Wq, Wk, Wv, Wo):
    B, T, D = x.shape
    DKV = Wq.shape[0]
    C = _CHUNK
    return pl.pallas_call(
        _fastkv_kernel,
        out_shape=jax.ShapeDtypeStruct((B, T, D), x.dtype),
        grid=(B, T // C),
        in_specs=[
            pl.BlockSpec((1, C, D), lambda b, c: (b, c, 0)),
            pl.BlockSpec((DKV, D), lambda b, c: (0, 0)),
            pl.BlockSpec((DKV, D), lambda b, c: (0, 0)),
            pl.BlockSpec((DKV, D), lambda b, c: (0, 0)),
            pl.BlockSpec((D, DKV), lambda b, c: (0, 0)),
        ],
        out_specs=pl.BlockSpec((1, C, D), lambda b, c: (b, c, 0)),
        scratch_shapes=[pltpu.VMEM((DKV, DKV), _F32)],
        compiler_params=pltpu.CompilerParams(
            dimension_semantics=("parallel", "arbitrary"),
        ),
        name="fastkv_chunked",
    )(x, Wq, Wk, Wv, Wo)
